# baseline (speedup 1.0000x reference)
import functools

import jax
import jax.numpy as jnp
from jax import lax
from jax.experimental import pallas as pl
from jax.experimental.pallas import tpu as pltpu
from jax.experimental.pallas import tpu_sc as plsc

B = 4096
HIST = 50
UF = 10
IF = 5
D = 32
NC = 2
NS = 16
NW = NC * NS
RPW = B // NW
HP = 256
UFP = 16
IFP = 8
U_IN = (UF + IF) * D
I_IN = IF * D


_LOG_VB = 16
_VB = 1 << _LOG_VB
_LOG_SUB = _LOG_VB - 2
_SUB = _VB // 4
_VOCAB = 1000000
_VGRID = (_VOCAB + _VB - 1) // _VB
_VPAD = _VGRID * _VB


def _tc_relayout(wt):
    def body(in_ref, out_ref):
        for q in range(4):
            x = in_ref[:, q * _SUB:(q + 1) * _SUB]
            out_ref[:, q * D:(q + 1) * D] = jnp.transpose(x)

    return pl.pallas_call(
        body,
        grid=(_VGRID,),
        in_specs=[pl.BlockSpec((D, _VB), lambda i: (0, i))],
        out_specs=pl.BlockSpec((_SUB, 128), lambda i: (i, 0)),
        out_shape=jax.ShapeDtypeStruct((_VPAD * D // 128, 128), jnp.float32),
    )(wt)


def _sc_gather_pool(W_emb, xu_t, xi_t, hist2, len3):
    mesh = plsc.VectorSubcoreMesh(core_axis_name="c", subcore_axis_name="s")

    @functools.partial(
        pl.kernel,
        out_type=[
            jax.ShapeDtypeStruct((B, U_IN), jnp.float32),
            jax.ShapeDtypeStruct((B, I_IN), jnp.float32),
        ],
        mesh=mesh,
        compiler_params=pltpu.CompilerParams(use_tc_tiling_on_sc=False,
                                              needs_layout_passes=False),
        scratch_types=[
            pltpu.VMEM((UFP, RPW), jnp.int32),
            pltpu.VMEM((IFP, RPW), jnp.int32),
            pltpu.VMEM((2 * RPW, 128), jnp.int32),
            pltpu.VMEM((8, 16), jnp.int32),
            pltpu.VMEM((HP, D), jnp.float32),
            pltpu.VMEM((HP, D), jnp.float32),
            pltpu.VMEM((IF, RPW, D), jnp.float32),
            pltpu.VMEM((IF, RPW, D), jnp.float32),
            pltpu.VMEM((RPW, I_IN), jnp.float32),
            pltpu.SemaphoreType.DMA,
            pltpu.SemaphoreType.DMA,
            pltpu.SemaphoreType.DMA,
        ],
    )
    def k(emb, xu, xi, hh, hl, u_out, i_out,
          uidx, iidx, hidx, lenv, hb0, hb1, ubuf, ibuf, pooled,
          sem0, sem1, sem_ui):
        wid = lax.axis_index("s") * NC + lax.axis_index("c")
        base = wid * RPW
        hb = (hb0, hb1)
        sems = (sem0, sem1)
        pltpu.sync_copy(xu.at[pl.ds(wid * UFP, UFP)], uidx)
        pltpu.sync_copy(xi.at[pl.ds(wid * IFP, IFP)], iidx)
        pltpu.sync_copy(hh.at[pl.ds(wid * 2 * RPW, 2 * RPW)], hidx)
        pltpu.sync_copy(hl.at[pl.ds(wid * 8, 8)], lenv)

        def gmap(v):
            return ((v >> _LOG_VB) << _LOG_VB) + ((v & (_SUB - 1)) << 2) + (
                (v >> _LOG_SUB) & 3)

        def xform_rows(ref, nrows):
            def xb(i, c):
                for j in range(8):
                    sl = pl.ds(j * 16, 16)
                    ref[i, sl] = gmap(ref[i, sl])
                return c
            lax.fori_loop(0, nrows, xb, 0)

        xform_rows(hidx, 2 * RPW)
        xform_rows(uidx, UFP)
        xform_rows(iidx, IFP)

        for f in range(IF):
            pltpu.async_copy(emb.at[uidx.at[f]], ubuf.at[f], sem_ui)
            pltpu.async_copy(emb.at[iidx.at[f]], ibuf.at[f], sem_ui)

        lane = lax.iota(jnp.int32, 16)

        def row_len(r):
            win = lenv[r // 16]
            return jnp.max(jnp.where(lane == (r % 16), win, 0))

        def hist_cps(r, b):
            cp0 = pltpu.make_async_copy(
                emb.at[hidx.at[2 * r]], hb[b].at[pl.ds(0, 128)], sems[b])
            cp1 = pltpu.make_async_copy(
                emb.at[hidx.at[2 * r + 1]], hb[b].at[pl.ds(128, 128)],
                sems[b])
            return cp0, cp1

        def fire(r, b):
            ls = row_len(r)
            cp0, cp1 = hist_cps(r, b)
            cp0.start()

            @pl.when(ls > 25)
            def _():
                cp1.start()

        def drain(r, b):
            ls = row_len(r)
            cp0, cp1 = hist_cps(r, b)
            cp0.wait()

            @pl.when(ls > 25)
            def _():
                cp1.wait()

        fire(0, 0)

        def g_body(g, carry):
            for b in range(2):
                r = 2 * g + b

                @pl.when(r < RPW - 1)
                def _():
                    fire(r + 1, 1 - b)

                drain(r, b)
                ls = row_len(r)
                lenvec = jnp.broadcast_to(ls, (16,))
                acc = tuple(jnp.zeros((16,), jnp.float32)
                            for _ in range(2 * IF))

                for blk in range(5):
                    def blk_fn(a, blk=blk, b=b):
                        a = list(a)
                        for dt in range(10):
                            t = blk * 10 + dt
                            m = lenvec > t
                            for f in range(IF):
                                p = t * IF + f
                                for h in range(2):
                                    v = hb[b][p, pl.ds(h * 16, 16)]
                                    a[2 * f + h] = a[2 * f + h] + jnp.where(
                                        m, v, 0.0)
                        return tuple(a)

                    acc = lax.cond(ls > blk * 10, blk_fn, lambda a: a, acc)

                for f in range(IF):
                    for h in range(2):
                        pooled[r, pl.ds(f * D + h * 16, 16)] = (
                            acc[2 * f + h] * (1.0 / HIST))
            return carry

        lax.fori_loop(0, RPW // 2, g_body, 0)

        for f in range(IF):
            pltpu.make_async_copy(emb.at[uidx.at[f]], ubuf.at[f],
                                  sem_ui).wait()
            pltpu.make_async_copy(emb.at[iidx.at[f]], ibuf.at[f],
                                  sem_ui).wait()
        outs = []
        for f in range(IF):
            outs.append(pltpu.async_copy(
                ubuf.at[f], u_out.at[pl.ds(base, RPW), pl.ds(f * D, D)],
                sem0))
            outs.append(pltpu.async_copy(
                ibuf.at[f], i_out.at[pl.ds(base, RPW), pl.ds(f * D, D)],
                sem0))
        for cp in outs:
            cp.wait()
        for f in range(IF):
            pltpu.async_copy(emb.at[uidx.at[IF + f]], ubuf.at[f], sem_ui)
        for f in range(IF):
            pltpu.make_async_copy(emb.at[uidx.at[IF + f]], ubuf.at[f],
                                  sem_ui).wait()
        outs = []
        for f in range(IF):
            outs.append(pltpu.async_copy(
                ubuf.at[f],
                u_out.at[pl.ds(base, RPW), pl.ds((IF + f) * D, D)], sem0))
        outs.append(pltpu.async_copy(
            pooled, u_out.at[pl.ds(base, RPW), pl.ds(UF * D, I_IN)], sem0))
        for cp in outs:
            cp.wait()

    return k(W_emb, xu_t, xi_t, hist2, len3)


def _towers(u_in, it_in, Wu1, bu1, Wu2, bu2, Wi1, bi1, Wi2, bi2):
    BB = 512

    def body(u_ref, it_ref, wu1, bu1r, wu2, bu2r, wi1, bi1r, wi2, bi2r,
             out_ref):
        u = jnp.tanh(jnp.dot(u_ref[...], wu1[...],
                             preferred_element_type=jnp.float32) + bu1r[...])
        u = jnp.tanh(jnp.dot(u, wu2[...],
                             preferred_element_type=jnp.float32) + bu2r[...])
        it = jnp.tanh(jnp.dot(it_ref[...], wi1[...],
                              preferred_element_type=jnp.float32) + bi1r[...])
        it = jnp.tanh(jnp.dot(it, wi2[...],
                              preferred_element_type=jnp.float32) + bi2r[...])
        out_ref[...] = jax.nn.sigmoid(
            jnp.sum(u * it, axis=1, keepdims=True))

    return pl.pallas_call(
        body,
        grid=(B // BB,),
        in_specs=[
            pl.BlockSpec((BB, U_IN), lambda i: (i, 0)),
            pl.BlockSpec((BB, I_IN), lambda i: (i, 0)),
            pl.BlockSpec((U_IN, 256), lambda i: (0, 0)),
            pl.BlockSpec((1, 256), lambda i: (0, 0)),
            pl.BlockSpec((256, 128), lambda i: (0, 0)),
            pl.BlockSpec((1, 128), lambda i: (0, 0)),
            pl.BlockSpec((I_IN, 256), lambda i: (0, 0)),
            pl.BlockSpec((1, 256), lambda i: (0, 0)),
            pl.BlockSpec((256, 128), lambda i: (0, 0)),
            pl.BlockSpec((1, 128), lambda i: (0, 0)),
        ],
        out_specs=pl.BlockSpec((BB, 1), lambda i: (i, 0)),
        out_shape=jax.ShapeDtypeStruct((B, 1), jnp.float32),
    )(u_in, it_in, Wu1, bu1.reshape(1, -1), Wu2, bu2.reshape(1, -1),
      Wi1, bi1.reshape(1, -1), Wi2, bi2.reshape(1, -1))


def kernel(W_emb, Wu1, bu1, Wu2, bu2, Wi1, bi1, Wi2, bi2,
           x_stat, x_user, x_item, user_hist, hist_len):
    xu_t = x_user.reshape(NW, RPW, UF).transpose(0, 2, 1)
    xu_t = jnp.pad(xu_t, ((0, 0), (0, UFP - UF), (0, 0))).reshape(NW * UFP, RPW)
    xi_t = x_item.reshape(NW, RPW, IF).transpose(0, 2, 1)
    xi_t = jnp.pad(xi_t, ((0, 0), (0, IFP - IF), (0, 0))).reshape(NW * IFP, RPW)
    hist2 = jnp.pad(user_hist.reshape(B, HIST * IF),
                    ((0, 0), (0, HP - HIST * IF))).reshape(2 * B, 128)
    len3 = hist_len.reshape(NW * 8, 16)
    W_lin = _tc_relayout(W_emb.T).reshape(_VPAD, D)
    u_in, it_in = _sc_gather_pool(W_lin, xu_t, xi_t, hist2, len3)
    score = _towers(u_in, it_in, Wu1, bu1, Wu2, bu2, Wi1, bi1, Wi2, bi2)
    return score.reshape(B)


# speedup vs baseline: 1.0151x; 1.0151x over previous
import functools

import jax
import jax.numpy as jnp
from jax import lax
from jax.experimental import pallas as pl
from jax.experimental.pallas import tpu as pltpu
from jax.experimental.pallas import tpu_sc as plsc

B = 4096
HIST = 50
UF = 10
IF = 5
D = 32
NC = 2
NS = 16
NW = NC * NS
RPW = B // NW
HP = 256
UFP = 16
IFP = 8
U_IN = (UF + IF) * D
I_IN = IF * D


_LOG_VB = 15
_VB = 1 << _LOG_VB
_LOG_SUB = _LOG_VB - 2
_SUB = _VB // 4
_VOCAB = 1000000
_VGRID = (_VOCAB + _VB - 1) // _VB
_VPAD = _VGRID * _VB


def _tc_relayout(wt):
    def body(in_ref, out_ref):
        for q in range(4):
            x = in_ref[:, q * _SUB:(q + 1) * _SUB]
            out_ref[:, q * D:(q + 1) * D] = jnp.transpose(x)

    return pl.pallas_call(
        body,
        grid=(_VGRID,),
        in_specs=[pl.BlockSpec((D, _VB), lambda i: (0, i))],
        out_specs=pl.BlockSpec((_SUB, 128), lambda i: (i, 0)),
        out_shape=jax.ShapeDtypeStruct((_VPAD * D // 128, 128), jnp.float32),
    )(wt)


def _sc_gather_pool(W_emb, xu_t, xi_t, hist2, len3):
    mesh = plsc.VectorSubcoreMesh(core_axis_name="c", subcore_axis_name="s")

    @functools.partial(
        pl.kernel,
        out_type=[
            jax.ShapeDtypeStruct((B, U_IN), jnp.float32),
            jax.ShapeDtypeStruct((B, I_IN), jnp.float32),
        ],
        mesh=mesh,
        compiler_params=pltpu.CompilerParams(use_tc_tiling_on_sc=False,
                                              needs_layout_passes=False),
        scratch_types=[
            pltpu.VMEM((UFP, RPW), jnp.int32),
            pltpu.VMEM((IFP, RPW), jnp.int32),
            pltpu.VMEM((2 * RPW, 128), jnp.int32),
            pltpu.VMEM((8, 16), jnp.int32),
            pltpu.VMEM((HP, D), jnp.float32),
            pltpu.VMEM((HP, D), jnp.float32),
            pltpu.VMEM((IF, RPW, D), jnp.float32),
            pltpu.VMEM((IF, RPW, D), jnp.float32),
            pltpu.VMEM((RPW, I_IN), jnp.float32),
            pltpu.SemaphoreType.DMA,
            pltpu.SemaphoreType.DMA,
            pltpu.SemaphoreType.DMA,
        ],
    )
    def k(emb, xu, xi, hh, hl, u_out, i_out,
          uidx, iidx, hidx, lenv, hb0, hb1, ubuf, ibuf, pooled,
          sem0, sem1, sem_ui):
        wid = lax.axis_index("s") * NC + lax.axis_index("c")
        base = wid * RPW
        hb = (hb0, hb1)
        sems = (sem0, sem1)
        pltpu.sync_copy(xu.at[pl.ds(wid * UFP, UFP)], uidx)
        pltpu.sync_copy(xi.at[pl.ds(wid * IFP, IFP)], iidx)
        pltpu.sync_copy(hh.at[pl.ds(wid * 2 * RPW, 2 * RPW)], hidx)
        pltpu.sync_copy(hl.at[pl.ds(wid * 8, 8)], lenv)

        def gmap(v):
            return ((v >> _LOG_VB) << _LOG_VB) + ((v & (_SUB - 1)) << 2) + (
                (v >> _LOG_SUB) & 3)

        def xform_rows(ref, nrows):
            def xb(i, c):
                for j in range(8):
                    sl = pl.ds(j * 16, 16)
                    ref[i, sl] = gmap(ref[i, sl])
                return c
            lax.fori_loop(0, nrows, xb, 0)

        xform_rows(hidx, 2 * RPW)
        xform_rows(uidx, UFP)
        xform_rows(iidx, IFP)

        for f in range(IF):
            pltpu.async_copy(emb.at[uidx.at[f]], ubuf.at[f], sem_ui)
            pltpu.async_copy(emb.at[iidx.at[f]], ibuf.at[f], sem_ui)

        lane = lax.iota(jnp.int32, 16)

        def row_len(r):
            win = lenv[r // 16]
            return jnp.max(jnp.where(lane == (r % 16), win, 0))

        def hist_cps(r, b):
            cp0 = pltpu.make_async_copy(
                emb.at[hidx.at[2 * r]], hb[b].at[pl.ds(0, 128)], sems[b])
            cp1 = pltpu.make_async_copy(
                emb.at[hidx.at[2 * r + 1]], hb[b].at[pl.ds(128, 128)],
                sems[b])
            return cp0, cp1

        def fire(r, b):
            ls = row_len(r)
            cp0, cp1 = hist_cps(r, b)
            cp0.start()

            @pl.when(ls > 25)
            def _():
                cp1.start()

        def drain(r, b):
            ls = row_len(r)
            cp0, cp1 = hist_cps(r, b)
            cp0.wait()

            @pl.when(ls > 25)
            def _():
                cp1.wait()

        fire(0, 0)

        def g_body(g, carry):
            for b in range(2):
                r = 2 * g + b

                @pl.when(r < RPW - 1)
                def _():
                    fire(r + 1, 1 - b)

                drain(r, b)
                ls = row_len(r)
                lenvec = jnp.broadcast_to(ls, (16,))
                acc = tuple(jnp.zeros((16,), jnp.float32)
                            for _ in range(2 * IF))

                for blk in range(5):
                    def blk_fn(a, blk=blk, b=b):
                        a = list(a)
                        for dt in range(10):
                            t = blk * 10 + dt
                            m = lenvec > t
                            for f in range(IF):
                                p = t * IF + f
                                for h in range(2):
                                    v = hb[b][p, pl.ds(h * 16, 16)]
                                    a[2 * f + h] = a[2 * f + h] + jnp.where(
                                        m, v, 0.0)
                        return tuple(a)

                    acc = lax.cond(ls > blk * 10, blk_fn, lambda a: a, acc)

                for f in range(IF):
                    for h in range(2):
                        pooled[r, pl.ds(f * D + h * 16, 16)] = (
                            acc[2 * f + h] * (1.0 / HIST))
            return carry

        lax.fori_loop(0, RPW // 2, g_body, 0)

        for f in range(IF):
            pltpu.make_async_copy(emb.at[uidx.at[f]], ubuf.at[f],
                                  sem_ui).wait()
            pltpu.make_async_copy(emb.at[iidx.at[f]], ibuf.at[f],
                                  sem_ui).wait()
        outs = []
        for f in range(IF):
            outs.append(pltpu.async_copy(
                ubuf.at[f], u_out.at[pl.ds(base, RPW), pl.ds(f * D, D)],
                sem0))
            outs.append(pltpu.async_copy(
                ibuf.at[f], i_out.at[pl.ds(base, RPW), pl.ds(f * D, D)],
                sem0))
        for cp in outs:
            cp.wait()
        for f in range(IF):
            pltpu.async_copy(emb.at[uidx.at[IF + f]], ubuf.at[f], sem_ui)
        for f in range(IF):
            pltpu.make_async_copy(emb.at[uidx.at[IF + f]], ubuf.at[f],
                                  sem_ui).wait()
        outs = []
        for f in range(IF):
            outs.append(pltpu.async_copy(
                ubuf.at[f],
                u_out.at[pl.ds(base, RPW), pl.ds((IF + f) * D, D)], sem0))
        outs.append(pltpu.async_copy(
            pooled, u_out.at[pl.ds(base, RPW), pl.ds(UF * D, I_IN)], sem0))
        for cp in outs:
            cp.wait()

    return k(W_emb, xu_t, xi_t, hist2, len3)


def _towers(u_in, it_in, Wu1, bu1, Wu2, bu2, Wi1, bi1, Wi2, bi2):
    BB = 512

    def body(u_ref, it_ref, wu1, bu1r, wu2, bu2r, wi1, bi1r, wi2, bi2r,
             out_ref):
        u = jnp.tanh(jnp.dot(u_ref[...], wu1[...],
                             preferred_element_type=jnp.float32) + bu1r[...])
        u = jnp.tanh(jnp.dot(u, wu2[...],
                             preferred_element_type=jnp.float32) + bu2r[...])
        it = jnp.tanh(jnp.dot(it_ref[...], wi1[...],
                              preferred_element_type=jnp.float32) + bi1r[...])
        it = jnp.tanh(jnp.dot(it, wi2[...],
                              preferred_element_type=jnp.float32) + bi2r[...])
        out_ref[...] = jax.nn.sigmoid(
            jnp.sum(u * it, axis=1, keepdims=True))

    return pl.pallas_call(
        body,
        grid=(B // BB,),
        in_specs=[
            pl.BlockSpec((BB, U_IN), lambda i: (i, 0)),
            pl.BlockSpec((BB, I_IN), lambda i: (i, 0)),
            pl.BlockSpec((U_IN, 256), lambda i: (0, 0)),
            pl.BlockSpec((1, 256), lambda i: (0, 0)),
            pl.BlockSpec((256, 128), lambda i: (0, 0)),
            pl.BlockSpec((1, 128), lambda i: (0, 0)),
            pl.BlockSpec((I_IN, 256), lambda i: (0, 0)),
            pl.BlockSpec((1, 256), lambda i: (0, 0)),
            pl.BlockSpec((256, 128), lambda i: (0, 0)),
            pl.BlockSpec((1, 128), lambda i: (0, 0)),
        ],
        out_specs=pl.BlockSpec((BB, 1), lambda i: (i, 0)),
        out_shape=jax.ShapeDtypeStruct((B, 1), jnp.float32),
    )(u_in, it_in, Wu1, bu1.reshape(1, -1), Wu2, bu2.reshape(1, -1),
      Wi1, bi1.reshape(1, -1), Wi2, bi2.reshape(1, -1))


def kernel(W_emb, Wu1, bu1, Wu2, bu2, Wi1, bi1, Wi2, bi2,
           x_stat, x_user, x_item, user_hist, hist_len):
    xu_t = x_user.reshape(NW, RPW, UF).transpose(0, 2, 1)
    xu_t = jnp.pad(xu_t, ((0, 0), (0, UFP - UF), (0, 0))).reshape(NW * UFP, RPW)
    xi_t = x_item.reshape(NW, RPW, IF).transpose(0, 2, 1)
    xi_t = jnp.pad(xi_t, ((0, 0), (0, IFP - IF), (0, 0))).reshape(NW * IFP, RPW)
    hist2 = jnp.pad(user_hist.reshape(B, HIST * IF),
                    ((0, 0), (0, HP - HIST * IF))).reshape(2 * B, 128)
    len3 = hist_len.reshape(NW * 8, 16)
    W_lin = _tc_relayout(W_emb.T).reshape(_VPAD, D)
    u_in, it_in = _sc_gather_pool(W_lin, xu_t, xi_t, hist2, len3)
    score = _towers(u_in, it_in, Wu1, bu1, Wu2, bu2, Wi1, bi1, Wi2, bi2)
    return score.reshape(B)
